# Initial kernel scaffold; baseline (speedup 1.0000x reference)
import jax
import jax.numpy as jnp
from jax.experimental import pallas as pl


def kernel(f_atoms, f_bonds, a2b, b2a, b2revb, W_i_atom, W_i_bond, W_h_0, W_h_1, W_lr, gru_bias, Wih_f, Whh_f, bih_f, bhh_f, Wih_b, Whh_b, bih_b, bhh_b, W_o, b_o, n_mols, atoms_per_mol):
    raise NotImplementedError("write your pallas kernel here")


# speedup vs baseline: 4.7088x; 4.7088x over previous
import functools

import jax
import jax.numpy as jnp
from jax import lax
from jax.experimental import pallas as pl
from jax.experimental.pallas import tpu as pltpu
from jax.experimental.pallas import tpu_sc as plsc

F32 = jnp.float32
_PC = pl.pallas_call
BF16 = jnp.bfloat16


def _dot(a, b):
    return jnp.dot(a.astype(BF16), b.astype(BF16), preferred_element_type=F32)

H = 300
HP = 384
BM = 512


def _rup(x, m):
    return ((x + m - 1) // m) * m


def _sc_gather(table, idx, chunk=96):
    B = idx.shape[0]
    V, D = table.shape
    NW = 32
    b_per_w = B // NW
    n_chunks = b_per_w // chunk
    mesh = plsc.VectorSubcoreMesh(core_axis_name="c", subcore_axis_name="s")

    @functools.partial(
        pl.kernel,
        mesh=mesh,
        out_type=jax.ShapeDtypeStruct((B, D), F32),
        scratch_types=[
            pltpu.VMEM((b_per_w,), jnp.int32),
            pltpu.VMEM((chunk, D), F32),
            pltpu.VMEM((chunk, D), F32),
            pltpu.SemaphoreType.DMA,
            pltpu.SemaphoreType.DMA,
        ],
    )
    def k(table_hbm, idx_hbm, out_hbm, idx_v, rows_v0, rows_v1, sem0, sem1):
        wid = lax.axis_index("s") * 2 + lax.axis_index("c")
        base = wid * b_per_w
        pltpu.sync_copy(idx_hbm.at[pl.ds(base, b_per_w)], idx_v)

        @pl.loop(0, n_chunks, step=2)
        def _(i):
            o0 = i * chunk
            o1 = o0 + chunk
            g0 = pltpu.make_async_copy(
                table_hbm.at[idx_v.at[pl.ds(o0, chunk)]], rows_v0, sem0)
            g0.start()
            g1 = pltpu.make_async_copy(
                table_hbm.at[idx_v.at[pl.ds(o1, chunk)]], rows_v1, sem1)
            g1.start()
            g0.wait()
            pltpu.sync_copy(rows_v0, out_hbm.at[pl.ds(base + o0, chunk)])
            g1.wait()
            pltpu.sync_copy(rows_v1, out_hbm.at[pl.ds(base + o1, chunk)])

    return k(table, idx)


def _mm_relu(x, wT):
    M, K = x.shape
    N = wT.shape[1]

    def body(x_ref, w_ref, o_ref):
        o_ref[...] = jnp.maximum(
            _dot(x_ref[...], w_ref[...]), 0.0)

    return _PC(
        body,
        grid=(M // BM,),
        in_specs=[pl.BlockSpec((BM, K), lambda i: (i, 0)),
                  pl.BlockSpec((K, N), lambda i: (0, 0))],
        out_specs=pl.BlockSpec((BM, N), lambda i: (i, 0)),
        out_shape=jax.ShapeDtypeStruct((M, N), F32),
    )(x, wT)


def _aggregate(nei, ma, add_ma):
    _, A, D = nei.shape

    def body(nei_ref, ma_ref, o_ref):
        acc = nei_ref[0]
        mx = nei_ref[0]
        for k in range(1, 6):
            v = nei_ref[k]
            acc = acc + v
            mx = jnp.maximum(mx, v)
        agg = acc * mx
        o_ref[...] = ma_ref[...] + agg if add_ma else agg

    return _PC(
        body,
        grid=(A // BM,),
        in_specs=[pl.BlockSpec((6, BM, D), lambda i: (0, i, 0)),
                  pl.BlockSpec((BM, D), lambda i: (i, 0))],
        out_specs=pl.BlockSpec((BM, D), lambda i: (i, 0)),
        out_shape=jax.ShapeDtypeStruct((A, D), F32),
    )(nei, ma)


def _bond_update(ib, am, rv, whT, nb_real):
    M, D = ib.shape

    def body(ib_ref, am_ref, rv_ref, w_ref, o_ref):
        i = pl.program_id(0)
        x = am_ref[...] - rv_ref[...]
        y = ib_ref[...] + _dot(x, w_ref[...])
        rows = i * BM + lax.broadcasted_iota(jnp.int32, (BM, 1), 0)
        o_ref[...] = jnp.where(rows < nb_real, jnp.maximum(y, 0.0), 0.0)

    return _PC(
        body,
        grid=(M // BM,),
        in_specs=[pl.BlockSpec((BM, D), lambda i: (i, 0)),
                  pl.BlockSpec((BM, D), lambda i: (i, 0)),
                  pl.BlockSpec((BM, D), lambda i: (i, 0)),
                  pl.BlockSpec((D, D), lambda i: (0, 0))],
        out_specs=pl.BlockSpec((BM, D), lambda i: (i, 0)),
        out_shape=jax.ShapeDtypeStruct((M, D), F32),
    )(ib, am, rv, whT)


def _lr_project(agg, ma, ia, wcat, gb):
    M, D = agg.shape
    N = wcat.shape[1]

    def body(a_ref, m_ref, i_ref, w_ref, gb_ref, hid_ref, msg_ref):
        cat = jnp.concatenate(
            [a_ref[:, :N], m_ref[:, :N], i_ref[:, :N]], axis=1)
        h = jnp.dot(cat, w_ref[...], preferred_element_type=F32)
        hid_ref[...] = h
        msg_ref[...] = jnp.maximum(h + gb_ref[...], 0.0)

    return _PC(
        body,
        grid=(M // BM,),
        in_specs=[pl.BlockSpec((BM, D), lambda i: (i, 0)),
                  pl.BlockSpec((BM, D), lambda i: (i, 0)),
                  pl.BlockSpec((BM, D), lambda i: (i, 0)),
                  pl.BlockSpec((3 * N, N), lambda i: (0, 0)),
                  pl.BlockSpec((1, N), lambda i: (0, 0))],
        out_specs=[pl.BlockSpec((BM, N), lambda i: (i, 0)),
                   pl.BlockSpec((BM, N), lambda i: (i, 0))],
        out_shape=[jax.ShapeDtypeStruct((M, N), F32),
                   jax.ShapeDtypeStruct((M, N), F32)],
    )(agg, ma, ia, wcat, gb)


def _seg_max(hid_m):
    nm, apm, D = hid_m.shape
    bm = 40

    def body(x_ref, o_ref):
        mx = x_ref[:, 0, :]
        for t in range(1, apm):
            mx = jnp.maximum(mx, x_ref[:, t, :])
        o_ref[...] = mx

    return _PC(
        body,
        grid=(nm // bm,),
        in_specs=[pl.BlockSpec((bm, apm, D), lambda i: (i, 0, 0))],
        out_specs=pl.BlockSpec((bm, D), lambda i: (i, 0)),
        out_shape=jax.ShapeDtypeStruct((nm, D), F32),
    )(hid_m)


def _bigru(xT, h0, wf, bf, wb, bb):
    T, B, D = xT.shape

    def body(xf_ref, xb_ref, h0_ref,
             firT, fizT, finT, fhrT, fhzT, fhnT, fbr, fbz, fbi, fbh,
             birT, bizT, binT, bhrT, bhzT, bhnT, bbr, bbz, bbi, bbh,
             of_ref, ob_ref, hf_s, hb_s):
        t = pl.program_id(0)

        @pl.when(t == 0)
        def _():
            hf_s[...] = h0_ref[...]
            hb_s[...] = h0_ref[...]

        def step(x, h, wirT, wizT, winT, whrT, whzT, whnT, br, bz, bi, bh):
            dot = lambda a, w: jnp.dot(a, w[...], preferred_element_type=F32)
            r = jax.nn.sigmoid(dot(x, wirT) + dot(h, whrT) + br[...])
            z = jax.nn.sigmoid(dot(x, wizT) + dot(h, whzT) + bz[...])
            n = jnp.tanh(dot(x, winT) + bi[...] + r * (dot(h, whnT) + bh[...]))
            return (1.0 - z) * n + z * h

        hf = step(xf_ref[0], hf_s[...], firT, fizT, finT, fhrT, fhzT, fhnT,
                  fbr, fbz, fbi, fbh)
        hf_s[...] = hf
        of_ref[0] = hf
        hb = step(xb_ref[0], hb_s[...], birT, bizT, binT, bhrT, bhzT, bhnT,
                  bbr, bbz, bbi, bbh)
        hb_s[...] = hb
        ob_ref[0] = hb

    w_spec = pl.BlockSpec((D, D), lambda t: (0, 0))
    b_spec = pl.BlockSpec((1, D), lambda t: (0, 0))
    return _PC(
        body,
        grid=(T,),
        in_specs=[pl.BlockSpec((1, B, D), lambda t: (t, 0, 0)),
                  pl.BlockSpec((1, B, D), lambda t: (T - 1 - t, 0, 0)),
                  pl.BlockSpec((B, D), lambda t: (0, 0))]
                 + [w_spec] * 6 + [b_spec] * 4
                 + [w_spec] * 6 + [b_spec] * 4,
        out_specs=[pl.BlockSpec((1, B, D), lambda t: (t, 0, 0)),
                   pl.BlockSpec((1, B, D), lambda t: (T - 1 - t, 0, 0))],
        out_shape=[jax.ShapeDtypeStruct((T, B, D), F32),
                   jax.ShapeDtypeStruct((T, B, D), F32)],
        scratch_shapes=[pltpu.VMEM((B, D), F32), pltpu.VMEM((B, D), F32)],
    )(xT, xT, h0, *wf, *bf, *wb, *bb)


def _out_project(out_f, out_b, wo, bo):
    T, B, D = out_f.shape

    def body(of_ref, ob_ref, w_ref, bo_ref, o_ref, acc_s):
        t = pl.program_id(0)
        cat = jnp.concatenate([of_ref[0], ob_ref[0]], axis=1)
        v = jnp.maximum(_dot(cat, w_ref[...]) + bo_ref[...], 0.0)

        @pl.when(t == 0)
        def _():
            acc_s[...] = v

        @pl.when(t > 0)
        def _():
            acc_s[...] = acc_s[...] + v

        @pl.when(t == T - 1)
        def _():
            o_ref[...] = acc_s[...] * (1.0 / T)

    return _PC(
        body,
        grid=(T,),
        in_specs=[pl.BlockSpec((1, B, D), lambda t: (t, 0, 0)),
                  pl.BlockSpec((1, B, D), lambda t: (t, 0, 0)),
                  pl.BlockSpec((2 * D, D), lambda t: (0, 0)),
                  pl.BlockSpec((1, D), lambda t: (0, 0))],
        out_specs=pl.BlockSpec((B, D), lambda t: (0, 0)),
        out_shape=jax.ShapeDtypeStruct((B, D), F32),
        scratch_shapes=[pltpu.VMEM((B, D), F32)],
    )(out_f, out_b, wo, bo)


def kernel(f_atoms, f_bonds, a2b, b2a, b2revb, W_i_atom, W_i_bond, W_h_0,
           W_h_1, W_lr, gru_bias, Wih_f, Whh_f, bih_f, bhh_f, Wih_b, Whh_b,
           bih_b, bhh_b, W_o, b_o, n_mols, atoms_per_mol):
    NA = f_atoms.shape[0]
    NB = f_bonds.shape[0]
    ZR = NB
    A_pad = _rup(NA, 1024)
    B_pad = _rup(NB + 1, 6144)
    apm = 50
    nm = (NA - 1) // apm

    fa = jnp.pad(f_atoms, ((0, A_pad - NA), (0, 0)))
    fb = jnp.pad(f_bonds, ((0, B_pad - NB), (0, 0)))
    a2b_eff = jnp.where(a2b == 0, ZR, a2b).astype(jnp.int32)
    idx_nei = jnp.pad(a2b_eff.T, ((0, 0), (0, A_pad - NA)),
                      constant_values=ZR).reshape(-1)
    idx_rev = jnp.pad(b2revb.astype(jnp.int32), (0, B_pad - NB))
    idx_b2a = jnp.pad(b2a.astype(jnp.int32), (0, B_pad - NB))

    def padT(w, cols=HP):
        wt = w.T
        return jnp.pad(wt, ((0, 0), (0, cols - wt.shape[1])))

    wiaT = padT(W_i_atom)
    wibT = padT(W_i_bond)
    whT = [jnp.pad(W.T, ((0, HP - H), (0, HP - H))) for W in (W_h_0, W_h_1)]
    wl = W_lr.T
    gb = gru_bias.reshape(1, H)

    def gru_mats(Wih, Whh, bih, bhh):
        wf = tuple(Wih[j * H:(j + 1) * H, :].T for j in range(3)) + \
             tuple(Whh[j * H:(j + 1) * H, :].T for j in range(3))
        bf = ((bih[:H] + bhh[:H]).reshape(1, H),
              (bih[H:2 * H] + bhh[H:2 * H]).reshape(1, H),
              bih[2 * H:].reshape(1, H), bhh[2 * H:].reshape(1, H))
        return wf, bf

    wf, bf = gru_mats(Wih_f, Whh_f, bih_f, bhh_f)
    wb, bb = gru_mats(Wih_b, Whh_b, bih_b, bhh_b)
    wo = W_o.T
    bo = b_o.reshape(1, H)

    input_atom = _mm_relu(fa, wiaT)
    input_bond = _mm_relu(fb, wibT)
    message_atom = input_atom
    message_bond = input_bond

    for r in range(2):
        nei = _sc_gather(message_bond, idx_nei).reshape(6, A_pad, HP)
        message_atom = _aggregate(nei, message_atom, add_ma=True)
        rev = _sc_gather(message_bond, idx_rev)
        am = _sc_gather(message_atom, idx_b2a)
        message_bond = _bond_update(input_bond, am, rev, whT[r], NB)

    nei = _sc_gather(message_bond, idx_nei).reshape(6, A_pad, HP)
    agg = _aggregate(nei, message_atom, add_ma=False)
    hid, msg = _lr_project(agg, message_atom, input_atom, wl, gb)

    hid_m = hid[1:1 + nm * apm].reshape(nm, apm, H)
    h0 = _seg_max(hid_m)
    xT = msg[1:1 + nm * apm].reshape(nm, apm, H).transpose(1, 0, 2)
    out_f, out_b = _bigru(xT, h0, wf, bf, wb, bb)
    return _out_project(out_f, out_b, wo, bo)
